# Initial kernel scaffold; baseline (speedup 1.0000x reference)
import functools

import jax
import jax.numpy as jnp
from jax import lax
from jax.experimental import pallas as pl
from jax.experimental.pallas import tpu as pltpu
from jax.experimental.pallas import tpu_sc as plsc

N = 100000
KVOL = 27
C = 16

NP = 102400
GPC = 16
CHUNK = GPC * 128
CPK = NP // CHUNK
NCHUNK = KVOL * CPK
NWORK = 32
ITERS = -(-NCHUNK // NWORK)
NACC = 100352
PTROWS = NACC // 16
DUMMY = N


def _tc_transform(in_feats, W):
    BM = 2500

    def body(x_ref, w_ref, o_ref):
        o_ref[0] = jnp.dot(x_ref[...], w_ref[0], preferred_element_type=jnp.float32)

    return pl.pallas_call(
        body,
        grid=(N // BM, KVOL),
        in_specs=[
            pl.BlockSpec((BM, C), lambda i, k: (i, 0)),
            pl.BlockSpec((1, C, C), lambda i, k: (k, 0, 0)),
        ],
        out_specs=pl.BlockSpec((1, BM, C), lambda i, k: (k, i, 0)),
        out_shape=jax.ShapeDtypeStruct((KVOL, N, C), jnp.float32),
    )(in_feats, W)


def _sc_gather_scatter(trans_flat, gidx, sidx3):
    mesh = plsc.VectorSubcoreMesh(core_axis_name="c", subcore_axis_name="s")

    @functools.partial(
        pl.kernel,
        out_type=jax.ShapeDtypeStruct((2, N, C), jnp.float32),
        mesh=mesh,
        scratch_types=[
            pltpu.VMEM((CHUNK,), jnp.int32),
            pltpu.VMEM((GPC, 128), jnp.int32),
            pltpu.VMEM((CHUNK, C), jnp.float32),
            pltpu.VMEM_SHARED((NACC, C), jnp.float32),
            pltpu.SemaphoreType.DMA,
        ],
    )
    def sck(trans_hbm, gidx_hbm, sidx_hbm, part_hbm, gbuf, sbuf, rows, acc, sem):
        cid = lax.axis_index("c")
        sid = lax.axis_index("s")
        wid = sid * 2 + cid

        def zb(i, _):
            rows[pl.ds(i, 1), :] = jnp.zeros((1, C), jnp.float32)
            return 0

        lax.fori_loop(0, CHUNK, zb, 0)
        zbase = sid * PTROWS
        for r in range(3):
            pltpu.sync_copy(rows, acc.at[pl.ds(zbase + r * CHUNK, CHUNK), :])
        pltpu.sync_copy(rows.at[pl.ds(0, 128), :],
                        acc.at[pl.ds(zbase + 3 * CHUNK, 128), :])
        plsc.subcore_barrier()

        def chunk_body(i, _):
            c = wid + NWORK * i

            @pl.when(c < NCHUNK)
            def _():
                k = c // CPK
                g0 = (c % CPK) * GPC
                pltpu.sync_copy(gidx_hbm.at[k, pl.ds(g0 * 128, CHUNK)], gbuf)
                pltpu.sync_copy(sidx_hbm.at[k, pl.ds(g0, GPC), :], sbuf)
                kN = k * N

                def adj(l, _):
                    gbuf[pl.ds(l * 16, 16)] = gbuf[pl.ds(l * 16, 16)] + kN
                    return 0

                lax.fori_loop(0, CHUNK // 16, adj, 0)

                gds = [
                    pltpu.async_copy(
                        trans_hbm.at[gbuf.at[pl.ds(j * 128, 128)]],
                        rows.at[pl.ds(j * 128, 128), :], sem)
                    for j in range(GPC)
                ]
                for d in gds:
                    d.wait()
                sds = [
                    pltpu.async_copy(
                        rows.at[pl.ds(j * 128, 128), :],
                        acc.at[sbuf.at[j]], sem, add=True)
                    for j in range(GPC)
                ]
                for d in sds:
                    d.wait()

            return 0

        lax.fori_loop(0, ITERS, chunk_body, 0)
        plsc.subcore_barrier()

        prows = N // 16
        pltpu.sync_copy(acc.at[pl.ds(sid * prows, prows), :],
                        part_hbm.at[cid, pl.ds(sid * prows, prows), :])

    return sck(trans_flat, gidx, sidx3)


def _tc_epilogue(parts, in_feats, b2):
    BM = 2500

    def body(p_ref, x_ref, b_ref, o_ref):
        s = p_ref[0] + p_ref[1] + b_ref[0]
        o_ref[...] = jnp.maximum(s, 0.0) * x_ref[...]

    return pl.pallas_call(
        body,
        grid=(N // BM,),
        in_specs=[
            pl.BlockSpec((2, BM, C), lambda i: (0, i, 0)),
            pl.BlockSpec((BM, C), lambda i: (i, 0)),
            pl.BlockSpec((1, C), lambda i: (0, 0)),
        ],
        out_specs=pl.BlockSpec((BM, C), lambda i: (i, 0)),
        out_shape=jax.ShapeDtypeStruct((N, C), jnp.float32),
    )(parts, in_feats, b2)


def kernel(in_feats, in_idx, out_idx, W, b):
    trans = _tc_transform(in_feats, W)
    gidx = jnp.pad(in_idx, ((0, 0), (0, NP - N)))
    sidx3 = jnp.pad(out_idx, ((0, 0), (0, NP - N)),
                    constant_values=DUMMY).reshape(KVOL, NP // 128, 128)
    parts = _sc_gather_scatter(trans.reshape(KVOL * N, C), gidx, sidx3)
    return _tc_epilogue(parts, in_feats, b.reshape(1, C))


# speedup vs baseline: 3.7624x; 3.7624x over previous
import functools

import jax
import jax.numpy as jnp
from jax import lax
from jax.experimental import pallas as pl
from jax.experimental.pallas import tpu as pltpu
from jax.experimental.pallas import tpu_sc as plsc

N = 100000
KVOL = 27
C = 16

NP = 102400
GPC = 16
CHUNK = GPC * 128
CPK = NP // CHUNK
NCHUNK = KVOL * CPK
ITERS = -(-NCHUNK // 16)
HALF = N // 2
NACC = 50176
PTROWS = NACC // 16
DUMMY = HALF


def _tc_transform(in_feats, W):
    BM = 2000

    def body(x_ref, w_ref, o_ref):
        o_ref[0] = jnp.dot(x_ref[...], w_ref[0], preferred_element_type=jnp.float32)

    return pl.pallas_call(
        body,
        grid=(N // BM, KVOL),
        in_specs=[
            pl.BlockSpec((BM, C), lambda i, k: (i, 0)),
            pl.BlockSpec((1, C, C), lambda i, k: (k, 0, 0)),
        ],
        out_specs=pl.BlockSpec((1, BM, C), lambda i, k: (k, i, 0)),
        out_shape=jax.ShapeDtypeStruct((KVOL, N, C), jnp.float32),
    )(in_feats, W)


def _sc_gather_scatter(trans_flat, gidx, sidx):
    mesh = plsc.VectorSubcoreMesh(core_axis_name="c", subcore_axis_name="s")

    @functools.partial(
        pl.kernel,
        out_type=jax.ShapeDtypeStruct((2, NACC, C), jnp.float32),
        mesh=mesh,
        scratch_types=[
            pltpu.VMEM((CHUNK,), jnp.int32),
            pltpu.VMEM((CHUNK,), jnp.int32),
            pltpu.VMEM((GPC, 128), jnp.int32),
            pltpu.VMEM((CHUNK, C), jnp.float32),
            pltpu.VMEM_SHARED((NACC, C), jnp.float32),
            pltpu.SemaphoreType.DMA,
        ],
        compiler_params=pltpu.CompilerParams(use_tc_tiling_on_sc=False),
    )
    def sck(trans_hbm, gidx_hbm, sidx_hbm, part_hbm, gbuf, tbuf, sbuf, rows,
            acc, sem):
        cid = lax.axis_index("c")
        sid = lax.axis_index("s")
        base = cid * HALF

        def zb(i, _):
            rows[pl.ds(i, 1), :] = jnp.zeros((1, C), jnp.float32)
            return 0

        lax.fori_loop(0, CHUNK, zb, 0)
        zbase = sid * PTROWS
        pltpu.sync_copy(rows, acc.at[pl.ds(zbase, CHUNK), :])
        pltpu.sync_copy(rows.at[pl.ds(0, 1088), :],
                        acc.at[pl.ds(zbase + CHUNK, 1088), :])
        plsc.subcore_barrier()

        def chunk_body(i, _):
            c = sid + 16 * i

            @pl.when(c < NCHUNK)
            def _():
                k = c // CPK
                g0 = (c % CPK) * GPC
                pltpu.sync_copy(gidx_hbm.at[k, pl.ds(g0 * 128, CHUNK)], gbuf)
                pltpu.sync_copy(sidx_hbm.at[k, pl.ds(g0 * 128, CHUNK)], tbuf)
                kN = k * N

                def adj(l, _):
                    gbuf[pl.ds(l * 16, 16)] = gbuf[pl.ds(l * 16, 16)] + kN
                    return 0

                lax.fori_loop(0, CHUNK // 16, adj, 0)

                for j in range(GPC):
                    def remap(l, _, j=j):
                        v = tbuf[pl.ds(j * 128 + l * 16, 16)]
                        w = v - base
                        m = (w >= 0) & (w < HALF)
                        sbuf[j, pl.ds(l * 16, 16)] = jnp.where(m, w, DUMMY)
                        return 0

                    lax.fori_loop(0, 8, remap, 0)

                gds = [
                    pltpu.async_copy(
                        trans_hbm.at[gbuf.at[pl.ds(j * 128, 128)]],
                        rows.at[pl.ds(j * 128, 128), :], sem)
                    for j in range(GPC)
                ]
                for d in gds:
                    d.wait()
                sds = [
                    pltpu.async_copy(
                        rows.at[pl.ds(j * 128, 128), :],
                        acc.at[sbuf.at[j]], sem, add=True)
                    for j in range(GPC)
                ]
                for d in sds:
                    d.wait()

            return 0

        lax.fori_loop(0, ITERS, chunk_body, 0)
        plsc.subcore_barrier()

        pltpu.sync_copy(acc.at[pl.ds(sid * PTROWS, PTROWS), :],
                        part_hbm.at[cid, pl.ds(sid * PTROWS, PTROWS), :])

    return sck(trans_flat, gidx, sidx)


def _tc_epilogue(parts, in_feats, b2):
    BM = 2000
    BPH = HALF // BM

    def body(p_ref, x_ref, b_ref, o_ref):
        s = p_ref[0] + b_ref[0]
        o_ref[...] = jnp.maximum(s, 0.0) * x_ref[...]

    return pl.pallas_call(
        body,
        grid=(N // BM,),
        in_specs=[
            pl.BlockSpec((1, BM, C), lambda i: (i // BPH, i % BPH, 0)),
            pl.BlockSpec((BM, C), lambda i: (i, 0)),
            pl.BlockSpec((1, C), lambda i: (0, 0)),
        ],
        out_specs=pl.BlockSpec((BM, C), lambda i: (i, 0)),
        out_shape=jax.ShapeDtypeStruct((N, C), jnp.float32),
    )(parts, in_feats, b2)


def kernel(in_feats, in_idx, out_idx, W, b):
    trans = _tc_transform(in_feats, W)
    gidx = jnp.pad(in_idx, ((0, 0), (0, NP - N)))
    sidx = jnp.pad(out_idx, ((0, 0), (0, NP - N)), constant_values=N)
    parts = _sc_gather_scatter(trans.reshape(KVOL * N, C), gidx, sidx)
    return _tc_epilogue(parts, in_feats, b.reshape(1, C))


# speedup vs baseline: 10.0016x; 2.6583x over previous
import functools

import jax
import jax.numpy as jnp
from jax import lax
from jax.experimental import pallas as pl
from jax.experimental.pallas import tpu as pltpu
from jax.experimental.pallas import tpu_sc as plsc

N = 100000
KVOL = 27
C = 16

NP = 102400
GPC = 16
CHUNK = GPC * 128
CPK = NP // CHUNK
NCHUNK = KVOL * CPK
ITERS = -(-NCHUNK // 16)
HALF = N // 2
NACC = 50176
PTROWS = NACC // 16
DUMMY = HALF


def _tc_transform(in_feats, W2):
    BM = 2000

    def body(x_ref, w_ref, o_ref):
        o_ref[...] = jnp.dot(x_ref[...], w_ref[...],
                             preferred_element_type=jnp.float32)

    return pl.pallas_call(
        body,
        grid=(N // BM,),
        in_specs=[
            pl.BlockSpec((BM, C), lambda i: (i, 0)),
            pl.BlockSpec((C, KVOL * C), lambda i: (0, 0)),
        ],
        out_specs=pl.BlockSpec((BM, KVOL * C), lambda i: (i, 0)),
        out_shape=jax.ShapeDtypeStruct((N, KVOL * C), jnp.float32),
    )(in_feats, W2)


def _sc_gather_scatter(trans_flat, gidx, sidx):
    mesh = plsc.VectorSubcoreMesh(core_axis_name="c", subcore_axis_name="s")

    @functools.partial(
        pl.kernel,
        out_type=jax.ShapeDtypeStruct((2, NACC, C), jnp.float32),
        mesh=mesh,
        scratch_types=[
            pltpu.VMEM((CHUNK,), jnp.int32),
            pltpu.VMEM((CHUNK,), jnp.int32),
            pltpu.VMEM((GPC, 128), jnp.int32),
            pltpu.VMEM((CHUNK, C), jnp.float32),
            pltpu.VMEM_SHARED((NACC, C), jnp.float32),
            pltpu.SemaphoreType.DMA,
        ],
        compiler_params=pltpu.CompilerParams(use_tc_tiling_on_sc=False),
    )
    def sck(trans_hbm, gidx_hbm, sidx_hbm, part_hbm, gbuf, tbuf, sbuf, rows,
            acc, sem):
        cid = lax.axis_index("c")
        sid = lax.axis_index("s")
        base = cid * HALF

        def zb(i, _):
            rows[pl.ds(i, 1), :] = jnp.zeros((1, C), jnp.float32)
            return 0

        lax.fori_loop(0, CHUNK, zb, 0)
        zbase = sid * PTROWS
        pltpu.sync_copy(rows, acc.at[pl.ds(zbase, CHUNK), :])
        pltpu.sync_copy(rows.at[pl.ds(0, 1088), :],
                        acc.at[pl.ds(zbase + CHUNK, 1088), :])
        plsc.subcore_barrier()

        def chunk_body(i, _):
            c = sid + 16 * i

            @pl.when(c < NCHUNK)
            def _():
                k = c // CPK
                g0 = (c % CPK) * GPC
                pltpu.sync_copy(gidx_hbm.at[k, pl.ds(g0 * 128, CHUNK)], gbuf)
                pltpu.sync_copy(sidx_hbm.at[k, pl.ds(g0 * 128, CHUNK)], tbuf)
                for j in range(GPC):
                    def fixup(l, _, j=j):
                        o = j * 128 + l * 16
                        gbuf[pl.ds(o, 16)] = gbuf[pl.ds(o, 16)] * KVOL + k
                        v = tbuf[pl.ds(o, 16)]
                        w = v - base
                        m = (w >= 0) & (w < HALF)
                        sbuf[j, pl.ds(l * 16, 16)] = jnp.where(
                            m, w, DUMMY + (v & 127))
                        return 0

                    lax.fori_loop(0, 8, fixup, 0)

                gds = [
                    pltpu.async_copy(
                        trans_hbm.at[gbuf.at[pl.ds(j * 128, 128)]],
                        rows.at[pl.ds(j * 128, 128), :], sem)
                    for j in range(GPC)
                ]
                for d in gds:
                    d.wait()
                sds = [
                    pltpu.async_copy(
                        rows.at[pl.ds(j * 128, 128), :],
                        acc.at[sbuf.at[j]], sem, add=True)
                    for j in range(GPC)
                ]
                for d in sds:
                    d.wait()

            return 0

        lax.fori_loop(0, ITERS, chunk_body, 0)
        plsc.subcore_barrier()

        pltpu.sync_copy(acc.at[pl.ds(sid * PTROWS, PTROWS), :],
                        part_hbm.at[cid, pl.ds(sid * PTROWS, PTROWS), :])

    return sck(trans_flat, gidx, sidx)


def _tc_epilogue(parts, in_feats, b2):
    BM = 2000
    BPH = HALF // BM

    def body(p_ref, x_ref, b_ref, o_ref):
        s = p_ref[0] + b_ref[0]
        o_ref[...] = jnp.maximum(s, 0.0) * x_ref[...]

    return pl.pallas_call(
        body,
        grid=(N // BM,),
        in_specs=[
            pl.BlockSpec((1, BM, C), lambda i: (i // BPH, i % BPH, 0)),
            pl.BlockSpec((BM, C), lambda i: (i, 0)),
            pl.BlockSpec((1, C), lambda i: (0, 0)),
        ],
        out_specs=pl.BlockSpec((BM, C), lambda i: (i, 0)),
        out_shape=jax.ShapeDtypeStruct((N, C), jnp.float32),
    )(parts, in_feats, b2)


def kernel(in_feats, in_idx, out_idx, W, b):
    W2 = W.transpose(1, 0, 2).reshape(C, KVOL * C)
    trans = _tc_transform(in_feats, W2)
    gidx = jnp.pad(in_idx, ((0, 0), (0, NP - N)))
    sidx = jnp.pad(out_idx, ((0, 0), (0, NP - N)), constant_values=N)
    parts = _sc_gather_scatter(trans.reshape(N * KVOL, C), gidx, sidx)
    return _tc_epilogue(parts, in_feats, b.reshape(1, C))


# speedup vs baseline: 10.3362x; 1.0335x over previous
import functools

import jax
import jax.numpy as jnp
from jax import lax
from jax.experimental import pallas as pl
from jax.experimental.pallas import tpu as pltpu
from jax.experimental.pallas import tpu_sc as plsc

N = 100000
KVOL = 27
C = 16

NP = 102400
GPC = 16
CHUNK = GPC * 128
CPK = NP // CHUNK
NCHUNK = KVOL * CPK
ITERS = -(-NCHUNK // 16)
HALF = N // 2
NACC = 50176
PTROWS = NACC // 16
DUMMY = HALF


def _tc_transform(in_feats, W2):
    BM = 2000

    def body(x_ref, w_ref, o_ref):
        o_ref[...] = jnp.dot(x_ref[...], w_ref[...],
                             preferred_element_type=jnp.float32)

    return pl.pallas_call(
        body,
        grid=(N // BM,),
        in_specs=[
            pl.BlockSpec((BM, C), lambda i: (i, 0)),
            pl.BlockSpec((C, KVOL * C), lambda i: (0, 0)),
        ],
        out_specs=pl.BlockSpec((BM, KVOL * C), lambda i: (i, 0)),
        out_shape=jax.ShapeDtypeStruct((N, KVOL * C), jnp.float32),
    )(in_feats, W2)


def _sc_gather_scatter(trans_flat, gidx, sidx):
    mesh = plsc.VectorSubcoreMesh(core_axis_name="c", subcore_axis_name="s")

    @functools.partial(
        pl.kernel,
        out_type=jax.ShapeDtypeStruct((2, NACC, C), jnp.float32),
        mesh=mesh,
        scratch_types=[
            pltpu.VMEM((CHUNK,), jnp.int32),
            pltpu.VMEM((CHUNK,), jnp.int32),
            pltpu.VMEM((CHUNK,), jnp.int32),
            pltpu.VMEM((CHUNK,), jnp.int32),
            pltpu.VMEM((GPC, 128), jnp.int32),
            pltpu.VMEM((GPC, 128), jnp.int32),
            pltpu.VMEM((CHUNK, C), jnp.float32),
            pltpu.VMEM((CHUNK, C), jnp.float32),
            pltpu.VMEM_SHARED((NACC, C), jnp.float32),
            pltpu.SemaphoreType.DMA,
            pltpu.SemaphoreType.DMA,
            pltpu.SemaphoreType.DMA,
            pltpu.SemaphoreType.DMA,
            pltpu.SemaphoreType.DMA,
            pltpu.SemaphoreType.DMA,
        ],
        compiler_params=pltpu.CompilerParams(use_tc_tiling_on_sc=False),
    )
    def sck(trans_hbm, gidx_hbm, sidx_hbm, part_hbm,
            gbufA, gbufB, tbufA, tbufB, sbufA, sbufB, rowsA, rowsB,
            acc, isemA, isemB, gsemA, gsemB, ssemA, ssemB):
        cid = lax.axis_index("c")
        sid = lax.axis_index("s")
        base = cid * HALF

        def zb(i, _):
            rowsA[pl.ds(i, 1), :] = jnp.zeros((1, C), jnp.float32)
            return 0

        lax.fori_loop(0, CHUNK, zb, 0)
        zbase = sid * PTROWS
        pltpu.sync_copy(rowsA, acc.at[pl.ds(zbase, CHUNK), :])
        pltpu.sync_copy(rowsA.at[pl.ds(0, 1088), :],
                        acc.at[pl.ds(zbase + CHUNK, 1088), :])
        plsc.subcore_barrier()

        def load_idx(c, gbuf, tbuf, isem):
            return [
                pltpu.async_copy(gidx_hbm.at[c], gbuf, isem),
                pltpu.async_copy(sidx_hbm.at[c], tbuf, isem),
            ]

        def fixup_idx(c, gbuf, tbuf, sbuf):
            k = c // CPK
            for j in range(GPC):
                def fx(l, _, j=j):
                    o = j * 128 + l * 16
                    gbuf[pl.ds(o, 16)] = gbuf[pl.ds(o, 16)] * KVOL + k
                    v = tbuf[pl.ds(o, 16)]
                    w = v - base
                    m = (w >= 0) & (w < HALF)
                    sbuf[j, pl.ds(l * 16, 16)] = jnp.where(
                        m, w, DUMMY + (v & 127))
                    return 0

                lax.fori_loop(0, 8, fx, 0)

        def fire_gathers(gbuf, rows, gsem):
            return [
                pltpu.async_copy(trans_hbm.at[gbuf.at[pl.ds(j * 128, 128)]],
                                 rows.at[pl.ds(j * 128, 128), :], gsem)
                for j in range(GPC)
            ]

        def fire_scatters(rows, sbuf, ssem):
            return [
                pltpu.async_copy(rows.at[pl.ds(j * 128, 128), :],
                                 acc.at[sbuf.at[j]], ssem, add=True)
                for j in range(GPC)
            ]

        def pair_body(ii, _):
            cA = sid + 16 * (2 * ii)
            cB = sid + 16 * (2 * ii + 1)
            onA = cA < NCHUNK
            onB = cB < NCHUNK
            dIA, dIB, dGA, dGB, dSA, dSB = [], [], [], [], [], []

            @pl.when(onA)
            def _():
                dIA.extend(load_idx(cA, gbufA, tbufA, isemA))

            @pl.when(onB)
            def _():
                dIB.extend(load_idx(cB, gbufB, tbufB, isemB))

            @pl.when(onA)
            def _():
                for d in dIA:
                    d.wait()
                fixup_idx(cA, gbufA, tbufA, sbufA)
                dGA.extend(fire_gathers(gbufA, rowsA, gsemA))

            @pl.when(onB)
            def _():
                for d in dIB:
                    d.wait()
                fixup_idx(cB, gbufB, tbufB, sbufB)

            @pl.when(onA)
            def _():
                for d in dGA:
                    d.wait()
                dSA.extend(fire_scatters(rowsA, sbufA, ssemA))

            @pl.when(onB)
            def _():
                dGB.extend(fire_gathers(gbufB, rowsB, gsemB))

            @pl.when(onA)
            def _():
                for d in dSA:
                    d.wait()

            @pl.when(onB)
            def _():
                for d in dGB:
                    d.wait()
                dSB.extend(fire_scatters(rowsB, sbufB, ssemB))
                for d in dSB:
                    d.wait()

            return 0

        lax.fori_loop(0, (ITERS + 1) // 2, pair_body, 0)
        plsc.subcore_barrier()

        pltpu.sync_copy(acc.at[pl.ds(sid * PTROWS, PTROWS), :],
                        part_hbm.at[cid, pl.ds(sid * PTROWS, PTROWS), :])

    return sck(trans_flat, gidx, sidx)


def _tc_epilogue(parts, in_feats, b2):
    BM = 2000
    BPH = HALF // BM

    def body(p_ref, x_ref, b_ref, o_ref):
        s = p_ref[0] + b_ref[0]
        o_ref[...] = jnp.maximum(s, 0.0) * x_ref[...]

    return pl.pallas_call(
        body,
        grid=(N // BM,),
        in_specs=[
            pl.BlockSpec((1, BM, C), lambda i: (i // BPH, i % BPH, 0)),
            pl.BlockSpec((BM, C), lambda i: (i, 0)),
            pl.BlockSpec((1, C), lambda i: (0, 0)),
        ],
        out_specs=pl.BlockSpec((BM, C), lambda i: (i, 0)),
        out_shape=jax.ShapeDtypeStruct((N, C), jnp.float32),
    )(parts, in_feats, b2)


def kernel(in_feats, in_idx, out_idx, W, b):
    W2 = W.transpose(1, 0, 2).reshape(C, KVOL * C)
    trans = _tc_transform(in_feats, W2)
    gidx = jnp.pad(in_idx, ((0, 0), (0, NP - N))).reshape(NCHUNK, CHUNK)
    sidx = jnp.pad(out_idx, ((0, 0), (0, NP - N)),
                   constant_values=N).reshape(NCHUNK, CHUNK)
    parts = _sc_gather_scatter(trans.reshape(N * KVOL, C), gidx, sidx)
    return _tc_epilogue(parts, in_feats, b.reshape(1, C))


# speedup vs baseline: 11.9390x; 1.1551x over previous
import functools

import jax
import jax.numpy as jnp
from jax import lax
from jax.experimental import pallas as pl
from jax.experimental.pallas import tpu as pltpu
from jax.experimental.pallas import tpu_sc as plsc

N = 100000
KVOL = 27
C = 16

NP = 102400
GPC = 16
CHUNK = GPC * 128
CPK = NP // CHUNK
NCHUNK = KVOL * CPK
CPC = NCHUNK // 2
ITERS = -(-CPC // 16)
NACC = 100128
PTROWS = NACC // 16
ZROWS = 2048


def _tc_transform(in_feats, W2):
    BM = 2000

    def body(x_ref, w_ref, o_ref):
        o_ref[...] = jnp.dot(x_ref[...], w_ref[...],
                             preferred_element_type=jnp.float32
                             ).astype(jnp.bfloat16)

    return pl.pallas_call(
        body,
        grid=(N // BM,),
        in_specs=[
            pl.BlockSpec((BM, C), lambda i: (i, 0)),
            pl.BlockSpec((C, KVOL * C), lambda i: (0, 0)),
        ],
        out_specs=pl.BlockSpec((BM, KVOL * C), lambda i: (i, 0)),
        out_shape=jax.ShapeDtypeStruct((N, KVOL * C), jnp.bfloat16),
    )(in_feats, W2)


def _sc_gather_scatter(trans_flat, gidx, sidx, zsrc):
    mesh = plsc.VectorSubcoreMesh(core_axis_name="c", subcore_axis_name="s")

    @functools.partial(
        pl.kernel,
        out_type=jax.ShapeDtypeStruct((2, NACC, C), jnp.bfloat16),
        mesh=mesh,
        scratch_types=[
            pltpu.VMEM((CHUNK,), jnp.int32),
            pltpu.VMEM((CHUNK,), jnp.int32),
            pltpu.VMEM((CHUNK,), jnp.int32),
            pltpu.VMEM((CHUNK,), jnp.int32),
            pltpu.VMEM((CHUNK, C), jnp.bfloat16),
            pltpu.VMEM((CHUNK, C), jnp.bfloat16),
            pltpu.VMEM_SHARED((NACC, C), jnp.bfloat16),
            pltpu.SemaphoreType.DMA,
            pltpu.SemaphoreType.DMA,
            pltpu.SemaphoreType.DMA,
            pltpu.SemaphoreType.DMA,
            pltpu.SemaphoreType.DMA,
            pltpu.SemaphoreType.DMA,
        ],
        compiler_params=pltpu.CompilerParams(use_tc_tiling_on_sc=False),
    )
    def sck(trans_hbm, gidx_hbm, sidx_hbm, zsrc_hbm, part_hbm,
            gbufA, gbufB, sbufA, sbufB, rowsA, rowsB,
            acc, isemA, isemB, gsemA, gsemB, ssemA, ssemB):
        cid = lax.axis_index("c")
        sid = lax.axis_index("s")

        zbase = sid * PTROWS
        for zo in range(0, PTROWS - ZROWS + 1, ZROWS):
            pltpu.sync_copy(zsrc_hbm,
                            acc.at[pl.ds(zbase + zo, ZROWS), :])
        ztail = PTROWS % ZROWS
        pltpu.sync_copy(zsrc_hbm.at[pl.ds(0, ztail), :],
                        acc.at[pl.ds(zbase + PTROWS - ztail, ztail), :])
        plsc.subcore_barrier()

        def load_idx(c, gbuf, sbuf, isem):
            return [
                pltpu.async_copy(gidx_hbm.at[c], gbuf, isem),
                pltpu.async_copy(sidx_hbm.at[c], sbuf, isem),
            ]

        def fire_gathers(gbuf, rows, gsem):
            return [
                pltpu.async_copy(trans_hbm.at[gbuf.at[pl.ds(j * 128, 128)]],
                                 rows.at[pl.ds(j * 128, 128), :], gsem)
                for j in range(GPC)
            ]

        def fire_scatters(rows, sbuf, ssem):
            return [
                pltpu.async_copy(rows.at[pl.ds(j * 128, 128), :],
                                 acc.at[sbuf.at[pl.ds(j * 128, 128)]],
                                 ssem, add=True)
                for j in range(GPC)
            ]

        base = cid * CPC

        def pair_body(ii, _):
            rA = sid + 16 * (2 * ii)
            rB = sid + 16 * (2 * ii + 1)
            cA = base + rA
            cB = base + rB
            onA = rA < CPC
            onB = rB < CPC
            dIA, dIB, dGA, dGB, dSA, dSB = [], [], [], [], [], []

            @pl.when(onA)
            def _():
                dIA.extend(load_idx(cA, gbufA, sbufA, isemA))

            @pl.when(onB)
            def _():
                dIB.extend(load_idx(cB, gbufB, sbufB, isemB))

            @pl.when(onA)
            def _():
                for d in dIA:
                    d.wait()
                dGA.extend(fire_gathers(gbufA, rowsA, gsemA))

            @pl.when(onB)
            def _():
                for d in dIB:
                    d.wait()

            @pl.when(onA)
            def _():
                for d in dGA:
                    d.wait()
                dSA.extend(fire_scatters(rowsA, sbufA, ssemA))

            @pl.when(onB)
            def _():
                dGB.extend(fire_gathers(gbufB, rowsB, gsemB))

            @pl.when(onA)
            def _():
                for d in dSA:
                    d.wait()

            @pl.when(onB)
            def _():
                for d in dGB:
                    d.wait()
                dSB.extend(fire_scatters(rowsB, sbufB, ssemB))
                for d in dSB:
                    d.wait()

            return 0

        lax.fori_loop(0, (ITERS + 1) // 2, pair_body, 0)
        plsc.subcore_barrier()

        pltpu.sync_copy(acc.at[pl.ds(sid * PTROWS, PTROWS), :],
                        part_hbm.at[cid, pl.ds(sid * PTROWS, PTROWS), :])

    return sck(trans_flat, gidx, sidx, zsrc)


def _tc_epilogue(parts, in_feats, b2):
    BM = 2000

    def body(p_ref, x_ref, b_ref, o_ref):
        s = (p_ref[0].astype(jnp.float32) + p_ref[1].astype(jnp.float32)
             + b_ref[0])
        o_ref[...] = jnp.maximum(s, 0.0) * x_ref[...]

    return pl.pallas_call(
        body,
        grid=(N // BM,),
        in_specs=[
            pl.BlockSpec((2, BM, C), lambda i: (0, i, 0)),
            pl.BlockSpec((BM, C), lambda i: (i, 0)),
            pl.BlockSpec((1, C), lambda i: (0, 0)),
        ],
        out_specs=pl.BlockSpec((BM, C), lambda i: (i, 0)),
        out_shape=jax.ShapeDtypeStruct((N, C), jnp.float32),
    )(parts, in_feats, b2)


def kernel(in_feats, in_idx, out_idx, W, b):
    W2 = W.transpose(1, 0, 2).reshape(C, KVOL * C)
    trans = _tc_transform(in_feats, W2)
    koff = jnp.arange(KVOL, dtype=jnp.int32)[:, None]
    gidx = jnp.pad(in_idx * KVOL + koff,
                   ((0, 0), (0, NP - N))).reshape(NCHUNK, CHUNK)
    padv = N + (jnp.arange(NP - N, dtype=jnp.int32) % 128)
    sidx = jnp.concatenate(
        [out_idx, jnp.broadcast_to(padv, (KVOL, NP - N))],
        axis=1).reshape(NCHUNK, CHUNK)
    zsrc = jnp.zeros((ZROWS, C), jnp.bfloat16)
    parts = _sc_gather_scatter(trans.reshape(N * KVOL, C), gidx, sidx, zsrc)
    return _tc_epilogue(parts, in_feats, b.reshape(1, C))


# speedup vs baseline: 12.2821x; 1.0287x over previous
import functools

import jax
import jax.numpy as jnp
from jax import lax
from jax.experimental import pallas as pl
from jax.experimental.pallas import tpu as pltpu
from jax.experimental.pallas import tpu_sc as plsc

N = 100000
KVOL = 27
C = 16
KP = 32
BM = 2000
NBLK = N // BM
JBLK = (KP * C) // 128

NP = 102400
GPC = 16
CHUNK = GPC * 128
CPK = NP // CHUNK
NCHUNK = KVOL * CPK
CPC = NCHUNK // 2
ITERS = -(-CPC // 16)
NACC = 100128
PTROWS = NACC // 16
ZROWS = 2048


def _tc_transform(in_feats, W2p):
    def body(x_ref, w_ref, o_ref):
        o_ref[...] = jnp.dot(x_ref[...], w_ref[...],
                             preferred_element_type=jnp.float32
                             ).astype(jnp.bfloat16)

    return pl.pallas_call(
        body,
        grid=(NBLK * JBLK,),
        in_specs=[
            pl.BlockSpec((BM, C), lambda g: (g // JBLK, 0)),
            pl.BlockSpec((C, 128), lambda g: (0, g % JBLK)),
        ],
        out_specs=pl.BlockSpec((BM, 128), lambda g: (g, 0)),
        out_shape=jax.ShapeDtypeStruct((N * JBLK, 128), jnp.bfloat16),
    )(in_feats, W2p)


def _sc_gather_scatter(trans_flat, gidx, sidx, zsrc):
    mesh = plsc.VectorSubcoreMesh(core_axis_name="c", subcore_axis_name="s")

    @functools.partial(
        pl.kernel,
        out_type=jax.ShapeDtypeStruct((2, NACC, C), jnp.bfloat16),
        mesh=mesh,
        scratch_types=[
            pltpu.VMEM((CHUNK,), jnp.int32),
            pltpu.VMEM((CHUNK,), jnp.int32),
            pltpu.VMEM((CHUNK,), jnp.int32),
            pltpu.VMEM((CHUNK,), jnp.int32),
            pltpu.VMEM((CHUNK, C), jnp.bfloat16),
            pltpu.VMEM((CHUNK, C), jnp.bfloat16),
            pltpu.VMEM_SHARED((NACC, C), jnp.bfloat16),
            pltpu.SemaphoreType.DMA,
            pltpu.SemaphoreType.DMA,
            pltpu.SemaphoreType.DMA,
            pltpu.SemaphoreType.DMA,
            pltpu.SemaphoreType.DMA,
            pltpu.SemaphoreType.DMA,
        ],
        compiler_params=pltpu.CompilerParams(use_tc_tiling_on_sc=False),
    )
    def sck(trans_hbm, gidx_hbm, sidx_hbm, zsrc_hbm, part_hbm,
            gbufA, gbufB, sbufA, sbufB, rowsA, rowsB,
            acc, isemA, isemB, gsemA, gsemB, ssemA, ssemB):
        cid = lax.axis_index("c")
        sid = lax.axis_index("s")

        zbase = sid * PTROWS
        for zo in range(0, PTROWS - ZROWS + 1, ZROWS):
            pltpu.sync_copy(zsrc_hbm,
                            acc.at[pl.ds(zbase + zo, ZROWS), :])
        ztail = PTROWS % ZROWS
        pltpu.sync_copy(zsrc_hbm.at[pl.ds(0, ztail), :],
                        acc.at[pl.ds(zbase + PTROWS - ztail, ztail), :])
        plsc.subcore_barrier()

        def load_idx(c, gbuf, sbuf, isem):
            return [
                pltpu.async_copy(gidx_hbm.at[c], gbuf, isem),
                pltpu.async_copy(sidx_hbm.at[c], sbuf, isem),
            ]

        def fire_gathers(gbuf, rows, gsem):
            return [
                pltpu.async_copy(trans_hbm.at[gbuf.at[pl.ds(j * 128, 128)]],
                                 rows.at[pl.ds(j * 128, 128), :], gsem)
                for j in range(GPC)
            ]

        def fire_scatters(rows, sbuf, ssem):
            return [
                pltpu.async_copy(rows.at[pl.ds(j * 128, 128), :],
                                 acc.at[sbuf.at[pl.ds(j * 128, 128)]],
                                 ssem, add=True)
                for j in range(GPC)
            ]

        base = cid * CPC

        def pair_body(ii, _):
            rA = sid + 16 * (2 * ii)
            rB = sid + 16 * (2 * ii + 1)
            cA = base + rA
            cB = base + rB
            onA = rA < CPC
            onB = rB < CPC
            dIA, dIB, dGA, dGB, dSA, dSB = [], [], [], [], [], []

            @pl.when(onA)
            def _():
                dIA.extend(load_idx(cA, gbufA, sbufA, isemA))

            @pl.when(onB)
            def _():
                dIB.extend(load_idx(cB, gbufB, sbufB, isemB))

            @pl.when(onA)
            def _():
                for d in dIA:
                    d.wait()
                dGA.extend(fire_gathers(gbufA, rowsA, gsemA))

            @pl.when(onB)
            def _():
                for d in dIB:
                    d.wait()

            @pl.when(onA)
            def _():
                for d in dGA:
                    d.wait()
                dSA.extend(fire_scatters(rowsA, sbufA, ssemA))

            @pl.when(onB)
            def _():
                dGB.extend(fire_gathers(gbufB, rowsB, gsemB))

            @pl.when(onA)
            def _():
                for d in dSA:
                    d.wait()

            @pl.when(onB)
            def _():
                for d in dGB:
                    d.wait()
                dSB.extend(fire_scatters(rowsB, sbufB, ssemB))
                for d in dSB:
                    d.wait()

            return 0

        lax.fori_loop(0, (ITERS + 1) // 2, pair_body, 0)
        plsc.subcore_barrier()

        pltpu.sync_copy(acc.at[pl.ds(sid * PTROWS, PTROWS), :],
                        part_hbm.at[cid, pl.ds(sid * PTROWS, PTROWS), :])

    return sck(trans_flat, gidx, sidx, zsrc)


def _tc_epilogue(parts, in_feats, b2):
    def body(p_ref, x_ref, b_ref, o_ref):
        s = (p_ref[0].astype(jnp.float32) + p_ref[1].astype(jnp.float32)
             + b_ref[...])
        o_ref[...] = jnp.maximum(s, 0.0) * x_ref[...]

    return pl.pallas_call(
        body,
        grid=(NBLK,),
        in_specs=[
            pl.BlockSpec((2, BM, C), lambda i: (0, i, 0)),
            pl.BlockSpec((BM, C), lambda i: (i, 0)),
            pl.BlockSpec((1, C), lambda i: (0, 0)),
        ],
        out_specs=pl.BlockSpec((BM, C), lambda i: (i, 0)),
        out_shape=jax.ShapeDtypeStruct((N, C), jnp.float32),
    )(parts, in_feats, b2)


def kernel(in_feats, in_idx, out_idx, W, b):
    W2 = W.transpose(1, 0, 2).reshape(C, KVOL * C)
    W2p = jnp.pad(W2, ((0, 0), (0, (KP - KVOL) * C)))
    trans128 = _tc_transform(in_feats, W2p)
    koff = jnp.arange(KVOL, dtype=jnp.int32)[:, None]
    gflat = ((in_idx // BM * JBLK + koff // 8) * BM + in_idx % BM) * 8 \
        + koff % 8
    gidx = jnp.pad(gflat, ((0, 0), (0, NP - N))).reshape(NCHUNK, CHUNK)
    padv = N + (jnp.arange(NP - N, dtype=jnp.int32) % 128)
    sidx = jnp.concatenate(
        [out_idx, jnp.broadcast_to(padv, (KVOL, NP - N))],
        axis=1).reshape(NCHUNK, CHUNK)
    zsrc = jnp.zeros((ZROWS, C), jnp.bfloat16)
    parts = _sc_gather_scatter(
        trans128.reshape(N * KP * C // 16, C), gidx, sidx, zsrc)
    return _tc_epilogue(parts, in_feats, b.reshape(1, C))


# speedup vs baseline: 12.3691x; 1.0071x over previous
import functools

import jax
import jax.numpy as jnp
from jax import lax
from jax.experimental import pallas as pl
from jax.experimental.pallas import tpu as pltpu
from jax.experimental.pallas import tpu_sc as plsc

N = 100000
KVOL = 27
C = 16
KP = 32
BM = 2000
NBLK = N // BM
JBLK = (KP * C) // 128

NP = 102400
GPC = 16
CHUNK = GPC * 128
CPK = NP // CHUNK
NCHUNK = KVOL * CPK
CPC = NCHUNK // 2
ITERS = -(-CPC // 16)
NACC = 100128
PTROWS = NACC // 16
ZROWS = 2048


def _tc_transform(in_feats, W2p):
    def body(x_ref, w_ref, o_ref):
        o_ref[...] = jnp.dot(x_ref[...], w_ref[...],
                             preferred_element_type=jnp.float32
                             ).astype(jnp.bfloat16)[None]

    return pl.pallas_call(
        body,
        grid=(NBLK * JBLK,),
        in_specs=[
            pl.BlockSpec((BM, C), lambda g: (g // JBLK, 0)),
            pl.BlockSpec((C, 128), lambda g: (0, g % JBLK)),
        ],
        out_specs=pl.BlockSpec((1, BM, 128), lambda g: (g % JBLK, g // JBLK, 0)),
        out_shape=jax.ShapeDtypeStruct((JBLK, N, 128), jnp.bfloat16),
    )(in_feats, W2p)


def _sc_gather_scatter(trans_flat, gidx, sidx, zsrc):
    mesh = plsc.VectorSubcoreMesh(core_axis_name="c", subcore_axis_name="s")

    @functools.partial(
        pl.kernel,
        out_type=jax.ShapeDtypeStruct((2, NACC, C), jnp.bfloat16),
        mesh=mesh,
        scratch_types=[
            pltpu.VMEM((CHUNK,), jnp.int32),
            pltpu.VMEM((CHUNK,), jnp.int32),
            pltpu.VMEM((CHUNK,), jnp.int32),
            pltpu.VMEM((CHUNK,), jnp.int32),
            pltpu.VMEM((CHUNK, C), jnp.bfloat16),
            pltpu.VMEM((CHUNK, C), jnp.bfloat16),
            pltpu.VMEM_SHARED((NACC, C), jnp.bfloat16),
            pltpu.SemaphoreType.DMA,
            pltpu.SemaphoreType.DMA,
            pltpu.SemaphoreType.DMA,
            pltpu.SemaphoreType.DMA,
            pltpu.SemaphoreType.DMA,
            pltpu.SemaphoreType.DMA,
        ],
        compiler_params=pltpu.CompilerParams(use_tc_tiling_on_sc=False),
    )
    def sck(trans_hbm, gidx_hbm, sidx_hbm, zsrc_hbm, part_hbm,
            gbufA, gbufB, sbufA, sbufB, rowsA, rowsB,
            acc, isemA, isemB, gsemA, gsemB, ssemA, ssemB):
        cid = lax.axis_index("c")
        sid = lax.axis_index("s")

        zbase = sid * PTROWS
        for zo in range(0, PTROWS - ZROWS + 1, ZROWS):
            pltpu.sync_copy(zsrc_hbm,
                            acc.at[pl.ds(zbase + zo, ZROWS), :])
        ztail = PTROWS % ZROWS
        pltpu.sync_copy(zsrc_hbm.at[pl.ds(0, ztail), :],
                        acc.at[pl.ds(zbase + PTROWS - ztail, ztail), :])
        plsc.subcore_barrier()

        def load_idx(c, gbuf, sbuf, isem):
            return [
                pltpu.async_copy(gidx_hbm.at[c], gbuf, isem),
                pltpu.async_copy(sidx_hbm.at[c], sbuf, isem),
            ]

        def fire_gathers(gbuf, rows, gsem):
            return [
                pltpu.async_copy(trans_hbm.at[gbuf.at[pl.ds(j * 128, 128)]],
                                 rows.at[pl.ds(j * 128, 128), :], gsem)
                for j in range(GPC)
            ]

        def fire_scatters(rows, sbuf, ssem):
            return [
                pltpu.async_copy(rows.at[pl.ds(j * 128, 128), :],
                                 acc.at[sbuf.at[pl.ds(j * 128, 128)]],
                                 ssem, add=True)
                for j in range(GPC)
            ]

        base = cid * CPC

        def pair_body(ii, _):
            rA = sid + 16 * (2 * ii)
            rB = sid + 16 * (2 * ii + 1)
            cA = base + rA
            cB = base + rB
            onA = rA < CPC
            onB = rB < CPC
            dIA, dIB, dGA, dGB, dSA, dSB = [], [], [], [], [], []

            @pl.when(onA)
            def _():
                dIA.extend(load_idx(cA, gbufA, sbufA, isemA))

            @pl.when(onB)
            def _():
                dIB.extend(load_idx(cB, gbufB, sbufB, isemB))

            @pl.when(onA)
            def _():
                for d in dIA:
                    d.wait()
                dGA.extend(fire_gathers(gbufA, rowsA, gsemA))

            @pl.when(onB)
            def _():
                for d in dIB:
                    d.wait()

            @pl.when(onA)
            def _():
                for d in dGA:
                    d.wait()
                dSA.extend(fire_scatters(rowsA, sbufA, ssemA))

            @pl.when(onB)
            def _():
                dGB.extend(fire_gathers(gbufB, rowsB, gsemB))

            @pl.when(onA)
            def _():
                for d in dSA:
                    d.wait()

            @pl.when(onB)
            def _():
                for d in dGB:
                    d.wait()
                dSB.extend(fire_scatters(rowsB, sbufB, ssemB))
                for d in dSB:
                    d.wait()

            return 0

        lax.fori_loop(0, (ITERS + 1) // 2, pair_body, 0)
        plsc.subcore_barrier()

        pltpu.sync_copy(acc.at[pl.ds(sid * PTROWS, PTROWS), :],
                        part_hbm.at[cid, pl.ds(sid * PTROWS, PTROWS), :])

    return sck(trans_flat, gidx, sidx, zsrc)


def _tc_epilogue(parts, in_feats, b2):
    def body(p_ref, x_ref, b_ref, o_ref):
        s = (p_ref[0].astype(jnp.float32) + p_ref[1].astype(jnp.float32)
             + b_ref[...])
        o_ref[...] = jnp.maximum(s, 0.0) * x_ref[...]

    return pl.pallas_call(
        body,
        grid=(NBLK,),
        in_specs=[
            pl.BlockSpec((2, BM, C), lambda i: (0, i, 0)),
            pl.BlockSpec((BM, C), lambda i: (i, 0)),
            pl.BlockSpec((1, C), lambda i: (0, 0)),
        ],
        out_specs=pl.BlockSpec((BM, C), lambda i: (i, 0)),
        out_shape=jax.ShapeDtypeStruct((N, C), jnp.float32),
    )(parts, in_feats, b2)


def kernel(in_feats, in_idx, out_idx, W, b):
    W2 = W.transpose(1, 0, 2).reshape(C, KVOL * C)
    W2p = jnp.pad(W2, ((0, 0), (0, (KP - KVOL) * C)))
    trans128 = _tc_transform(in_feats, W2p)
    koff = jnp.arange(KVOL, dtype=jnp.int32)[:, None]
    gflat = (in_idx << 3) + ((koff >> 3) * (8 * N) + (koff & 7))
    gidx = jnp.pad(gflat, ((0, 0), (0, NP - N))).reshape(NCHUNK, CHUNK)
    padv = N + (jnp.arange(NP - N, dtype=jnp.int32) % 128)
    sidx = jnp.concatenate(
        [out_idx, jnp.broadcast_to(padv, (KVOL, NP - N))],
        axis=1).reshape(NCHUNK, CHUNK)
    zsrc = jnp.zeros((ZROWS, C), jnp.bfloat16)
    parts = _sc_gather_scatter(
        trans128.reshape(N * KP * C // 16, C), gidx, sidx, zsrc)
    return _tc_epilogue(parts, in_feats, b.reshape(1, C))


# speedup vs baseline: 13.0027x; 1.0512x over previous
import functools

import jax
import jax.numpy as jnp
from jax import lax
from jax.experimental import pallas as pl
from jax.experimental.pallas import tpu as pltpu
from jax.experimental.pallas import tpu_sc as plsc

N = 100000
KVOL = 27
C = 16
KP = 32
BM = 2000
NBLK = N // BM
JBLK = (KP * C) // 128

NP = 102400
GPC = 16
CHUNK = GPC * 128
CPK = NP // CHUNK
NCHUNK = KVOL * CPK
CPC = NCHUNK // 2
ITERS = -(-CPC // 16)
NACC = 100128
PTROWS = NACC // 16
ZROWS = 2048


def _tc_transform(in_feats, W2p):
    def body(x_ref, w_ref, o_ref):
        o_ref[...] = jnp.dot(x_ref[...], w_ref[...],
                             preferred_element_type=jnp.float32
                             ).astype(jnp.bfloat16)[None]

    return pl.pallas_call(
        body,
        grid=(NBLK * JBLK,),
        in_specs=[
            pl.BlockSpec((BM, C), lambda g: (g // JBLK, 0)),
            pl.BlockSpec((C, 128), lambda g: (0, g % JBLK)),
        ],
        out_specs=pl.BlockSpec((1, BM, 128), lambda g: (g % JBLK, g // JBLK, 0)),
        out_shape=jax.ShapeDtypeStruct((JBLK, N, 128), jnp.bfloat16),
    )(in_feats, W2p)


def _sc_gather_scatter(trans_flat, gidx, sidx, zsrc):
    mesh = plsc.VectorSubcoreMesh(core_axis_name="c", subcore_axis_name="s")

    @functools.partial(
        pl.kernel,
        out_type=jax.ShapeDtypeStruct((2, NACC, C), jnp.bfloat16),
        mesh=mesh,
        scratch_types=[
            pltpu.VMEM((CHUNK,), jnp.int32),
            pltpu.VMEM((CHUNK,), jnp.int32),
            pltpu.VMEM((CHUNK,), jnp.int32),
            pltpu.VMEM((CHUNK,), jnp.int32),
            pltpu.VMEM((CHUNK, C), jnp.bfloat16),
            pltpu.VMEM((CHUNK, C), jnp.bfloat16),
            pltpu.VMEM_SHARED((NACC, C), jnp.bfloat16),
            pltpu.SemaphoreType.DMA,
            pltpu.SemaphoreType.DMA,
            pltpu.SemaphoreType.DMA,
            pltpu.SemaphoreType.DMA,
            pltpu.SemaphoreType.DMA,
            pltpu.SemaphoreType.DMA,
        ],
        compiler_params=pltpu.CompilerParams(use_tc_tiling_on_sc=False),
    )
    def sck(trans_hbm, gidx_hbm, sidx_hbm, zsrc_hbm, part_hbm,
            gbufA, gbufB, sbufA, sbufB, rowsA, rowsB,
            acc, isemA, isemB, gsemA, gsemB, ssemA, ssemB):
        cid = lax.axis_index("c")
        sid = lax.axis_index("s")

        zbase = sid * PTROWS
        for zo in range(0, PTROWS - ZROWS + 1, ZROWS):
            pltpu.sync_copy(zsrc_hbm,
                            acc.at[pl.ds(zbase + zo, ZROWS), :])
        ztail = PTROWS % ZROWS
        pltpu.sync_copy(zsrc_hbm.at[pl.ds(0, ztail), :],
                        acc.at[pl.ds(zbase + PTROWS - ztail, ztail), :])
        plsc.subcore_barrier()

        def load_idx(c, gbuf, sbuf, isem):
            return [
                pltpu.async_copy(gidx_hbm.at[c], gbuf, isem),
                pltpu.async_copy(sidx_hbm.at[c], sbuf, isem),
            ]

        def fire_gathers(gbuf, rows, gsem):
            return [
                pltpu.async_copy(trans_hbm.at[gbuf.at[pl.ds(j * 128, 128)]],
                                 rows.at[pl.ds(j * 128, 128), :], gsem)
                for j in range(GPC)
            ]

        def fire_scatters(rows, sbuf, ssem):
            return [
                pltpu.async_copy(rows.at[pl.ds(j * 128, 128), :],
                                 acc.at[sbuf.at[pl.ds(j * 128, 128)]],
                                 ssem, add=True)
                for j in range(GPC)
            ]

        base = cid * CPC

        def pair_body(ii, _):
            rA = sid + 16 * (2 * ii)
            rB = sid + 16 * (2 * ii + 1)
            cA = base + rA
            cB = base + rB
            onA = rA < CPC
            onB = rB < CPC
            dIA, dIB, dGA, dGB, dSA, dSB = [], [], [], [], [], []

            @pl.when(onA)
            def _():
                dIA.extend(load_idx(cA, gbufA, sbufA, isemA))

            @pl.when(onB)
            def _():
                dIB.extend(load_idx(cB, gbufB, sbufB, isemB))

            @pl.when(onA)
            def _():
                for d in dIA:
                    d.wait()
                dGA.extend(fire_gathers(gbufA, rowsA, gsemA))

            @pl.when(onB)
            def _():
                for d in dIB:
                    d.wait()

            @pl.when(onA)
            def _():
                for d in dGA:
                    d.wait()
                dSA.extend(fire_scatters(rowsA, sbufA, ssemA))

            @pl.when(onB)
            def _():
                dGB.extend(fire_gathers(gbufB, rowsB, gsemB))

            @pl.when(onA)
            def _():
                for d in dSA:
                    d.wait()

            @pl.when(onB)
            def _():
                for d in dGB:
                    d.wait()
                dSB.extend(fire_scatters(rowsB, sbufB, ssemB))
                for d in dSB:
                    d.wait()

            return 0

        lax.fori_loop(0, (ITERS + 1) // 2, pair_body, 0)
        plsc.subcore_barrier()

        pltpu.sync_copy(acc.at[pl.ds(sid * PTROWS, PTROWS), :],
                        part_hbm.at[cid, pl.ds(sid * PTROWS, PTROWS), :])

    return sck(trans_flat, gidx, sidx, zsrc)


def _tc_epilogue(parts_v, in128, b128):
    def body(p_ref, x_ref, b_ref, o_ref):
        p0 = p_ref[0][: N // 8].astype(jnp.float32)
        p1 = p_ref[1][: N // 8].astype(jnp.float32)
        s = p0 + p1 + b_ref[...]
        o_ref[...] = jnp.maximum(s, 0.0) * x_ref[...]

    return pl.pallas_call(
        body,
        grid=(1,),
        in_specs=[
            pl.BlockSpec((2, NACC // 8, 128), lambda i: (0, 0, 0)),
            pl.BlockSpec((N // 8, 128), lambda i: (0, 0)),
            pl.BlockSpec((1, 128), lambda i: (0, 0)),
        ],
        out_specs=pl.BlockSpec((N // 8, 128), lambda i: (0, 0)),
        out_shape=jax.ShapeDtypeStruct((N // 8, 128), jnp.float32),
    )(parts_v, in128, b128)


def kernel(in_feats, in_idx, out_idx, W, b):
    W2 = W.transpose(1, 0, 2).reshape(C, KVOL * C)
    W2p = jnp.pad(W2, ((0, 0), (0, (KP - KVOL) * C)))
    trans128 = _tc_transform(in_feats, W2p)
    koff = jnp.arange(KVOL, dtype=jnp.int32)[:, None]
    gflat = (in_idx << 3) + ((koff >> 3) * (8 * N) + (koff & 7))
    gidx = jnp.pad(gflat, ((0, 0), (0, NP - N))).reshape(NCHUNK, CHUNK)
    padv = N + (jnp.arange(NP - N, dtype=jnp.int32) % 128)
    sidx = jnp.concatenate(
        [out_idx, jnp.broadcast_to(padv, (KVOL, NP - N))],
        axis=1).reshape(NCHUNK, CHUNK)
    zsrc = jnp.zeros((ZROWS, C), jnp.bfloat16)
    parts = _sc_gather_scatter(
        trans128.reshape(N * KP * C // 16, C), gidx, sidx, zsrc)
    out128 = _tc_epilogue(parts.reshape(2, NACC // 8, 128),
                          in_feats.reshape(N // 8, 128),
                          jnp.tile(b, 8).reshape(1, 128))
    return out128.reshape(N, C)


# speedup vs baseline: 13.0051x; 1.0002x over previous
import functools

import jax
import jax.numpy as jnp
from jax import lax
from jax.experimental import pallas as pl
from jax.experimental.pallas import tpu as pltpu
from jax.experimental.pallas import tpu_sc as plsc

N = 100000
KVOL = 27
C = 16
KP = 32
BM = 2000
NBLK = N // BM
JBLK = (KP * C) // 128

NP = 102400
GPC = 16
CHUNK = GPC * 128
CPK = NP // CHUNK
NCHUNK = KVOL * CPK
CPC = NCHUNK // 2
ITERS = -(-CPC // 16)
NACC = 100128
PTROWS = NACC // 16
ZROWS = 2048


def _tc_transform(in_feats, W2p):
    def body(x_ref, w_ref, o_ref):
        o_ref[...] = jnp.dot(x_ref[...], w_ref[...],
                             preferred_element_type=jnp.float32
                             ).astype(jnp.bfloat16)[None]

    return pl.pallas_call(
        body,
        grid=(NBLK * JBLK,),
        in_specs=[
            pl.BlockSpec((BM, C), lambda g: (g // JBLK, 0)),
            pl.BlockSpec((C, 128), lambda g: (0, g % JBLK)),
        ],
        out_specs=pl.BlockSpec((1, BM, 128), lambda g: (g % JBLK, g // JBLK, 0)),
        out_shape=jax.ShapeDtypeStruct((JBLK, N, 128), jnp.bfloat16),
    )(in_feats, W2p)


def _sc_gather_scatter(trans_flat, gidx, sidx, zsrc):
    mesh = plsc.VectorSubcoreMesh(core_axis_name="c", subcore_axis_name="s")

    @functools.partial(
        pl.kernel,
        out_type=jax.ShapeDtypeStruct((2, NACC, C), jnp.bfloat16),
        mesh=mesh,
        scratch_types=[
            pltpu.VMEM((GPC, 128), jnp.int32),
            pltpu.VMEM((GPC, 128), jnp.int32),
            pltpu.VMEM((GPC, 128), jnp.int32),
            pltpu.VMEM((GPC, 128), jnp.int32),
            pltpu.VMEM((CHUNK, C), jnp.bfloat16),
            pltpu.VMEM((CHUNK, C), jnp.bfloat16),
            pltpu.VMEM_SHARED((NACC, C), jnp.bfloat16),
            pltpu.SemaphoreType.DMA,
            pltpu.SemaphoreType.DMA,
            pltpu.SemaphoreType.DMA,
            pltpu.SemaphoreType.DMA,
            pltpu.SemaphoreType.DMA,
            pltpu.SemaphoreType.DMA,
        ],
        compiler_params=pltpu.CompilerParams(use_tc_tiling_on_sc=False),
    )
    def sck(trans_hbm, gidx_hbm, sidx_hbm, zsrc_hbm, part_hbm,
            gbufA, gbufB, sbufA, sbufB, rowsA, rowsB,
            acc, isemA, isemB, gsemA, gsemB, ssemA, ssemB):
        cid = lax.axis_index("c")
        sid = lax.axis_index("s")

        zbase = sid * PTROWS
        for zo in range(0, PTROWS - ZROWS + 1, ZROWS):
            pltpu.sync_copy(zsrc_hbm,
                            acc.at[pl.ds(zbase + zo, ZROWS), :])
        ztail = PTROWS % ZROWS
        pltpu.sync_copy(zsrc_hbm.at[pl.ds(0, ztail), :],
                        acc.at[pl.ds(zbase + PTROWS - ztail, ztail), :])
        plsc.subcore_barrier()

        def load_idx(row, gbuf, sbuf, isem):
            return [
                pltpu.async_copy(gidx_hbm.at[pl.ds(row, GPC), :],
                                 gbuf, isem),
                pltpu.async_copy(sidx_hbm.at[pl.ds(row, GPC), :],
                                 sbuf, isem),
            ]

        def fire_gathers(gbuf, rows, gsem):
            return [
                pltpu.async_copy(trans_hbm.at[gbuf.at[j]],
                                 rows.at[pl.ds(j * 128, 128), :], gsem)
                for j in range(GPC)
            ]

        def fire_scatters(rows, sbuf, ssem):
            return [
                pltpu.async_copy(rows.at[pl.ds(j * 128, 128), :],
                                 acc.at[sbuf.at[j]],
                                 ssem, add=True)
                for j in range(GPC)
            ]

        base = cid * CPC

        def pair_body(ii, _):
            rA = sid + 16 * (2 * ii)
            rB = sid + 16 * (2 * ii + 1)
            rowA = (base + rA) * GPC
            rowB = (base + rB) * GPC
            onA = rA < CPC
            onB = rB < CPC
            dIA, dIB, dGA, dGB, dSA, dSB = [], [], [], [], [], []

            @pl.when(onA)
            def _():
                dIA.extend(load_idx(rowA, gbufA, sbufA, isemA))

            @pl.when(onB)
            def _():
                dIB.extend(load_idx(rowB, gbufB, sbufB, isemB))

            @pl.when(onA)
            def _():
                for d in dIA:
                    d.wait()
                dGA.extend(fire_gathers(gbufA, rowsA, gsemA))

            @pl.when(onB)
            def _():
                for d in dIB:
                    d.wait()

            @pl.when(onA)
            def _():
                for d in dGA:
                    d.wait()
                dSA.extend(fire_scatters(rowsA, sbufA, ssemA))

            @pl.when(onB)
            def _():
                dGB.extend(fire_gathers(gbufB, rowsB, gsemB))

            @pl.when(onA)
            def _():
                for d in dSA:
                    d.wait()

            @pl.when(onB)
            def _():
                for d in dGB:
                    d.wait()
                dSB.extend(fire_scatters(rowsB, sbufB, ssemB))
                for d in dSB:
                    d.wait()

            return 0

        lax.fori_loop(0, (ITERS + 1) // 2, pair_body, 0)
        plsc.subcore_barrier()

        pltpu.sync_copy(acc.at[pl.ds(sid * PTROWS, PTROWS), :],
                        part_hbm.at[cid, pl.ds(sid * PTROWS, PTROWS), :])

    return sck(trans_flat, gidx, sidx, zsrc)


def _tc_epilogue(parts_v, in128, b128):
    def body(p_ref, x_ref, b_ref, o_ref):
        p0 = p_ref[0][: N // 8].astype(jnp.float32)
        p1 = p_ref[1][: N // 8].astype(jnp.float32)
        s = p0 + p1 + b_ref[...]
        o_ref[...] = jnp.maximum(s, 0.0) * x_ref[...]

    return pl.pallas_call(
        body,
        grid=(1,),
        in_specs=[
            pl.BlockSpec((2, NACC // 8, 128), lambda i: (0, 0, 0)),
            pl.BlockSpec((N // 8, 128), lambda i: (0, 0)),
            pl.BlockSpec((1, 128), lambda i: (0, 0)),
        ],
        out_specs=pl.BlockSpec((N // 8, 128), lambda i: (0, 0)),
        out_shape=jax.ShapeDtypeStruct((N // 8, 128), jnp.float32),
    )(parts_v, in128, b128)


def kernel(in_feats, in_idx, out_idx, W, b):
    W2 = W.transpose(1, 0, 2).reshape(C, KVOL * C)
    W2p = jnp.pad(W2, ((0, 0), (0, (KP - KVOL) * C)))
    trans128 = _tc_transform(in_feats, W2p)
    koff = jnp.arange(KVOL, dtype=jnp.int32)[:, None]
    gflat = (in_idx << 3) + ((koff >> 3) * (8 * N) + (koff & 7))
    gidx = jnp.pad(gflat, ((0, 0), (0, NP - N))).reshape(NCHUNK * GPC, 128)
    padv = N + (jnp.arange(NP - N, dtype=jnp.int32) % 128)
    sidx = jnp.concatenate(
        [out_idx, jnp.broadcast_to(padv, (KVOL, NP - N))],
        axis=1).reshape(NCHUNK * GPC, 128)
    zsrc = jnp.zeros((ZROWS, C), jnp.bfloat16)
    parts = _sc_gather_scatter(
        trans128.reshape(N * KP * C // 16, C), gidx, sidx, zsrc)
    out128 = _tc_epilogue(parts.reshape(2, NACC // 8, 128),
                          in_feats.reshape(N // 8, 128),
                          jnp.tile(b, 8).reshape(1, 128))
    return out128.reshape(N, C)
